# baseline (speedup 1.0000x reference)
import functools

import jax
import jax.numpy as jnp
from jax import lax
from jax.experimental import pallas as pl
from jax.experimental.pallas import tpu as pltpu
from jax.experimental.pallas import tpu_sc as plsc

BATCH = 4096
SEQ = 200
EMBED = 64
NC, NS = 2, 16
NW = NC * NS
BPW = BATCH // NW

_JUNROLL = 8
_NBUF = 4


def _accum(buf, zacc):
    def body(jj, carry):
        for u in range(_JUNROLL):
            j = jj * _JUNROLL + u
            for c in range(4):
                plsc.addupdate(
                    zacc.at[j, pl.ds(16 * c, 16)], buf[j, pl.ds(16 * c, 16)]
                )
        return carry

    lax.fori_loop(0, BPW // _JUNROLL, body, 0)


def _pool_body(
    idx_hbm, table_hbm, z_hbm, idx_v, buf0, buf1, buf2, buf3, zacc,
    sem0, sem1, sem2, sem3,
):
    bufs = (buf0, buf1, buf2, buf3)
    sems = (sem0, sem1, sem2, sem3)
    wid = lax.axis_index("s") * NC + lax.axis_index("c")
    pltpu.sync_copy(idx_hbm.at[wid], idx_v)

    zero = jnp.zeros((16,), jnp.float32)

    def zbody(j, carry):
        for c in range(4):
            zacc[j, pl.ds(16 * c, 16)] = zero
        return carry

    lax.fori_loop(0, BPW, zbody, 0)

    for b in range(_NBUF):
        pltpu.async_copy(table_hbm.at[idx_v.at[b]], bufs[b], sems[b])

    def group(k, carry):
        s = _NBUF * k
        for b in range(_NBUF):
            pltpu.make_async_copy(
                table_hbm.at[idx_v.at[s + b]], bufs[b], sems[b]
            ).wait()
            _accum(bufs[b], zacc)

            @pl.when(s + b + _NBUF < SEQ)
            def _():
                pltpu.async_copy(
                    table_hbm.at[idx_v.at[s + b + _NBUF]], bufs[b], sems[b]
                )

        return carry

    lax.fori_loop(0, SEQ // _NBUF, group, 0)

    scale = jnp.float32(1.0 / SEQ)

    def sbody(j, carry):
        for c in range(4):
            zacc[j, pl.ds(16 * c, 16)] = zacc[j, pl.ds(16 * c, 16)] * scale
        return carry

    lax.fori_loop(0, BPW, sbody, 0)
    pltpu.sync_copy(zacc, z_hbm.at[pl.ds(wid * BPW, BPW)])


@jax.jit
def _pool(idx3, table_rm):
    mesh = plsc.VectorSubcoreMesh(core_axis_name="c", subcore_axis_name="s")
    kern = pl.kernel(
        _pool_body,
        out_type=jax.ShapeDtypeStruct((BATCH, EMBED), jnp.float32),
        mesh=mesh,
        scratch_types=[
            pltpu.VMEM((SEQ, 128), jnp.int32),
            pltpu.VMEM((128, EMBED), jnp.float32),
            pltpu.VMEM((128, EMBED), jnp.float32),
            pltpu.VMEM((128, EMBED), jnp.float32),
            pltpu.VMEM((128, EMBED), jnp.float32),
            pltpu.VMEM((BPW, EMBED), jnp.float32),
            pltpu.SemaphoreType.DMA,
            pltpu.SemaphoreType.DMA,
            pltpu.SemaphoreType.DMA,
            pltpu.SemaphoreType.DMA,
        ],
        compiler_params=pltpu.CompilerParams(use_tc_tiling_on_sc=False),
    )
    return kern(idx3, table_rm)


_VCHUNK = 32768


_HCHUNK = _VCHUNK // 2
_TSUB = 4096


def _tr_body(tT_ref, o_ref):
    for k in range(_VCHUNK // _TSUB):
        tt = jnp.transpose(tT_ref[:, pl.ds(k * _TSUB, _TSUB)])
        half, r0 = divmod(k * _TSUB, _HCHUNK)
        o_ref[pl.ds(r0, _TSUB), pl.ds(half * EMBED, EMBED)] = tt


def _transpose_pack(tableT):
    vocab = tableT.shape[1]
    grid = pl.cdiv(vocab, _VCHUNK)
    out = pl.pallas_call(
        _tr_body,
        grid=(grid,),
        in_specs=[pl.BlockSpec((EMBED, _VCHUNK), lambda i: (0, i))],
        out_specs=pl.BlockSpec((_HCHUNK, 128), lambda i: (i, 0)),
        out_shape=jax.ShapeDtypeStruct((grid * _HCHUNK, 128), jnp.float32),
    )(tableT)
    return out.reshape(grid * _VCHUNK, EMBED)


def _dense_body(z_ref, w1_ref, b1_ref, w2_ref, b2_ref, o_ref):
    z1 = jnp.dot(z_ref[...], w1_ref[...], preferred_element_type=jnp.float32)
    z1 = z1 + b1_ref[...]
    z2 = jnp.dot(z1, w2_ref[...], preferred_element_type=jnp.float32)
    o_ref[...] = z2 + b2_ref[...]


def kernel(x, table, W1, b1, W2, b2):
    l = x & (_VCHUNK - 1)
    xm = (x - l) + 2 * (l & (_HCHUNK - 1)) + (l >> (_HCHUNK.bit_length() - 1))
    idx3 = xm.reshape(NW, BPW, SEQ).swapaxes(1, 2)
    table_rm = _transpose_pack(table.T)
    z = _pool(idx3, table_rm)
    out = pl.pallas_call(
        _dense_body,
        out_shape=jax.ShapeDtypeStruct((BATCH, W2.shape[1]), jnp.float32),
    )(z, W1, b1.reshape(1, -1), W2, b2.reshape(1, -1))
    return out


# speedup vs baseline: 1.0847x; 1.0847x over previous
import functools

import jax
import jax.numpy as jnp
from jax import lax
from jax.experimental import pallas as pl
from jax.experimental.pallas import tpu as pltpu
from jax.experimental.pallas import tpu_sc as plsc

BATCH = 4096
SEQ = 200
EMBED = 64
NC, NS = 2, 16
NW = NC * NS
BPW = BATCH // NW

_JUNROLL = 8
_NBUF = 4


_GRP = 2


def _accum(buf, zacc):
    def body(jj, carry):
        for u in range(_JUNROLL):
            j = jj * _JUNROLL + u
            for c in range(4):
                d = pl.ds(16 * c, 16)
                plsc.addupdate(zacc.at[j, d], buf[j, d] + buf[BPW + j, d])
        return carry

    lax.fori_loop(0, BPW // _JUNROLL, body, 0)


def _pool_body(
    idx_hbm, table_hbm, z_hbm, idx_v, buf0, buf1, buf2, buf3, zacc,
    sem0, sem1, sem2, sem3,
):
    bufs = (buf0, buf1, buf2, buf3)
    sems = (sem0, sem1, sem2, sem3)
    wid = lax.axis_index("s") * NC + lax.axis_index("c")
    pltpu.sync_copy(idx_hbm.at[wid], idx_v)

    zero = jnp.zeros((16,), jnp.float32)

    def zbody(j, carry):
        for c in range(4):
            zacc[j, pl.ds(16 * c, 16)] = zero
        return carry

    lax.fori_loop(0, BPW, zbody, 0)

    ngrp = SEQ // _GRP

    def _fill(buf, sem, g):
        for h in range(_GRP):
            pltpu.async_copy(
                table_hbm.at[idx_v.at[_GRP * g + h]],
                buf.at[pl.ds(BPW * h, BPW)],
                sem,
            )

    for b in range(_NBUF):
        _fill(bufs[b], sems[b], b)

    def group(k, carry):
        for b in range(_NBUF):
            g = _NBUF * k + b
            for h in range(_GRP):
                pltpu.make_async_copy(
                    table_hbm.at[idx_v.at[_GRP * g + h]],
                    bufs[b].at[pl.ds(BPW * h, BPW)],
                    sems[b],
                ).wait()
            _accum(bufs[b], zacc)

            @pl.when(g + _NBUF < ngrp)
            def _():
                _fill(bufs[b], sems[b], g + _NBUF)

        return carry

    lax.fori_loop(0, ngrp // _NBUF, group, 0)

    scale = jnp.float32(1.0 / SEQ)

    def sbody(j, carry):
        for c in range(4):
            zacc[j, pl.ds(16 * c, 16)] = zacc[j, pl.ds(16 * c, 16)] * scale
        return carry

    lax.fori_loop(0, BPW, sbody, 0)
    pltpu.sync_copy(zacc, z_hbm.at[pl.ds(wid * BPW, BPW)])


@jax.jit
def _pool(idx3, table_rm):
    mesh = plsc.VectorSubcoreMesh(core_axis_name="c", subcore_axis_name="s")
    kern = pl.kernel(
        _pool_body,
        out_type=jax.ShapeDtypeStruct((BATCH, EMBED), jnp.float32),
        mesh=mesh,
        scratch_types=[
            pltpu.VMEM((SEQ, 128), jnp.int32),
            pltpu.VMEM((_GRP * 128, EMBED), jnp.float32),
            pltpu.VMEM((_GRP * 128, EMBED), jnp.float32),
            pltpu.VMEM((_GRP * 128, EMBED), jnp.float32),
            pltpu.VMEM((_GRP * 128, EMBED), jnp.float32),
            pltpu.VMEM((BPW, EMBED), jnp.float32),
            pltpu.SemaphoreType.DMA,
            pltpu.SemaphoreType.DMA,
            pltpu.SemaphoreType.DMA,
            pltpu.SemaphoreType.DMA,
        ],
        compiler_params=pltpu.CompilerParams(use_tc_tiling_on_sc=False),
    )
    return kern(idx3, table_rm)


_VCHUNK = 32768


_HCHUNK = _VCHUNK // 2
_TSUB = 4096


def _tr_body(tT_ref, o_ref):
    for k in range(_VCHUNK // _TSUB):
        tt = jnp.transpose(tT_ref[:, pl.ds(k * _TSUB, _TSUB)])
        half, r0 = divmod(k * _TSUB, _HCHUNK)
        o_ref[pl.ds(r0, _TSUB), pl.ds(half * EMBED, EMBED)] = tt


def _transpose_pack(tableT):
    vocab = tableT.shape[1]
    grid = pl.cdiv(vocab, _VCHUNK)
    out = pl.pallas_call(
        _tr_body,
        grid=(grid,),
        in_specs=[pl.BlockSpec((EMBED, _VCHUNK), lambda i: (0, i))],
        out_specs=pl.BlockSpec((_HCHUNK, 128), lambda i: (i, 0)),
        out_shape=jax.ShapeDtypeStruct((grid * _HCHUNK, 128), jnp.float32),
    )(tableT)
    return out.reshape(grid * _VCHUNK, EMBED)


def _dense_body(z_ref, w1_ref, b1_ref, w2_ref, b2_ref, o_ref):
    z1 = jnp.dot(z_ref[...], w1_ref[...], preferred_element_type=jnp.float32)
    z1 = z1 + b1_ref[...]
    z2 = jnp.dot(z1, w2_ref[...], preferred_element_type=jnp.float32)
    o_ref[...] = z2 + b2_ref[...]


def kernel(x, table, W1, b1, W2, b2):
    l = x & (_VCHUNK - 1)
    xm = (x - l) + 2 * (l & (_HCHUNK - 1)) + (l >> (_HCHUNK.bit_length() - 1))
    idx3 = xm.reshape(NW, BPW, SEQ).swapaxes(1, 2)
    table_rm = _transpose_pack(table.T)
    z = _pool(idx3, table_rm)
    out = pl.pallas_call(
        _dense_body,
        out_shape=jax.ShapeDtypeStruct((BATCH, W2.shape[1]), jnp.float32),
    )(z, W1, b1.reshape(1, -1), W2, b2.reshape(1, -1))
    return out
